# baseline (speedup 1.0000x reference)
import functools

import numpy as np
import jax
import jax.numpy as jnp
from jax import lax
from jax.experimental import pallas as pl
from jax.experimental.pallas import tpu as pltpu
from jax.experimental.pallas import tpu_sc as plsc

_K0 = np.uint32(0)
_K1 = np.uint32(1)
_K2 = np.uint32(int(_K0) ^ int(_K1) ^ 0x1BD11BDA)
_ROT = ((13, 15, 26, 6), (17, 29, 16, 24))

_B, _H, _W = 64, 1024, 512
_RB = 256
_GRID = (_B, _H // _RB)

_U_LO = np.float32(np.nextafter(np.float32(-1.0), np.float32(0.0)))
_C = np.float32(2.5580564)
_PG = (-0.0063512907, 0.0011399789, 0.04059056, -0.06747421,
       0.03346732, 1.491577, 3.1025813)


def _threefry_bits(x1):
    ks = (_K0, _K1, _K2)
    x0 = x1
    x1 = ((x1 << np.uint32(13)) | (x1 >> np.uint32(19))) ^ x0
    for i in range(5):
        for r in _ROT[i % 2][1 if i == 0 else 0:]:
            x0 = x0 + x1
            x1 = ((x1 << np.uint32(r)) | (x1 >> np.uint32(32 - r))) ^ x0
        if int(ks[(i + 1) % 3]) != 0:
            x0 = x0 + ks[(i + 1) % 3]
        x1 = x1 + np.uint32((int(ks[(i + 2) % 3]) + i + 1) & 0xFFFFFFFF)
    return x0 ^ x1


def _std_normal(idx):
    bits = _threefry_bits(idx)
    f = jax.lax.bitcast_convert_type(
        (bits >> np.uint32(9)) | np.uint32(0x40000000), jnp.float32)
    u = jnp.maximum(_U_LO, f - np.float32(3.0))
    s = jnp.sqrt(np.float32(1.0) - jnp.log(np.float32(1.0) - u * u))
    y = s - _C
    p = jnp.full_like(y, _PG[0])
    for c in _PG[1:]:
        p = p * y + np.float32(c)
    return u * p


_RC = 128


_SC_MESH = plsc.VectorSubcoreMesh(core_axis_name="c", subcore_axis_name="s")


@functools.partial(
    pl.kernel,
    mesh=_SC_MESH,
    out_type=jax.ShapeDtypeStruct((_B,), jnp.float32),
    scratch_types=[
        pltpu.VMEM((_B,), jnp.int32),
        pltpu.VMEM((_B,), jnp.float32),
        pltpu.SemaphoreType.DMA,
    ],
)
def _sc_gather(t_hbm, a_hbm, out_hbm, idx_v, vals_v, sem):
    @pl.when((lax.axis_index("c") == 0) & (lax.axis_index("s") == 0))
    def _():
        pltpu.sync_copy(t_hbm, idx_v)
        pltpu.async_copy(a_hbm.at[idx_v], vals_v, sem).wait()
        pltpu.sync_copy(vals_v, out_hbm)


def _noise_kernel(abar_ref, x0_ref, xt_ref, eps_ref):
    b = pl.program_id(0)
    rb = pl.program_id(1)
    abar = abar_ref[b]
    s0 = jnp.sqrt(abar)
    s1 = jnp.sqrt(np.float32(1.0) - abar)
    base1 = (b * np.int32(_H * _W) + rb * np.int32(_RB * _W)
             + np.int32(int(_K1))).astype(jnp.uint32)
    row = jax.lax.broadcasted_iota(jnp.uint32, (1, _RC, _W), 1)
    col = jax.lax.broadcasted_iota(jnp.uint32, (1, _RC, _W), 2)
    local = row * np.uint32(_W) + col

    def body(c, _):
        sl = pl.ds(c * _RC, _RC)
        idx = (base1 + (c * np.int32(_RC * _W)).astype(jnp.uint32)) + local
        eps = _std_normal(idx)
        eps_ref[:, sl, :] = eps
        xt_ref[:, sl, :] = s0 * x0_ref[:, sl, :] + s1 * eps
        return _

    jax.lax.fori_loop(0, _RB // _RC, body, 0, unroll=False)


def kernel(x0, t, alphas_cumprod):
    abar = _sc_gather(t, alphas_cumprod)
    blk = pl.BlockSpec((1, _RB, _W), lambda b, r: (b, r, 0))
    out = jax.ShapeDtypeStruct((_B, _H, _W), jnp.float32)
    x_t, eps = pl.pallas_call(
        _noise_kernel,
        grid=_GRID,
        in_specs=[
            pl.BlockSpec(memory_space=pltpu.SMEM),
            blk,
        ],
        out_specs=[blk, blk],
        out_shape=[out, out],
    )(abar, x0)
    return (x_t, eps)


# speedup vs baseline: 1.0128x; 1.0128x over previous
import functools

import numpy as np
import jax
import jax.numpy as jnp
from jax import lax
from jax.experimental import pallas as pl
from jax.experimental.pallas import tpu as pltpu
from jax.experimental.pallas import tpu_sc as plsc

_K0 = np.uint32(0)
_K1 = np.uint32(1)
_K2 = np.uint32(int(_K0) ^ int(_K1) ^ 0x1BD11BDA)
_ROT = ((13, 15, 26, 6), (17, 29, 16, 24))

_B, _H, _W = 64, 1024, 512
_RB = 256
_GRID = (_B, _H // _RB)

_U_LO = np.float32(np.nextafter(np.float32(-1.0), np.float32(0.0)))
_C = np.float32(0.62147385)
_PG = (462.2859191894531, -179.91043090820312, -48.60850524902344,
       3.161179304122925, 22.437999725341797, -14.577473640441895,
       7.130136013031006, -2.9441678524017334,
       1.802288293838501)


def _threefry_bits(x1):
    ks = (_K0, _K1, _K2)
    x0 = x1
    x1 = ((x1 << np.uint32(13)) | (x1 >> np.uint32(19))) ^ x0
    for i in range(5):
        for r in _ROT[i % 2][1 if i == 0 else 0:]:
            x0 = x0 + x1
            x1 = ((x1 << np.uint32(r)) | (x1 >> np.uint32(32 - r))) ^ x0
        if int(ks[(i + 1) % 3]) != 0:
            x0 = x0 + ks[(i + 1) % 3]
        x1 = x1 + np.uint32((int(ks[(i + 2) % 3]) + i + 1) & 0xFFFFFFFF)
    return x0 ^ x1


def _std_normal(idx):
    bits = _threefry_bits(idx)
    f = jax.lax.bitcast_convert_type(
        (bits >> np.uint32(9)) | np.uint32(0x40000000), jnp.float32)
    u = jnp.maximum(_U_LO, f - np.float32(3.0))
    z = jax.lax.rsqrt(np.float32(1.0) - jnp.log(np.float32(1.0) - u * u))
    y = z - _C
    p = jnp.full_like(y, _PG[0])
    for c in _PG[1:]:
        p = p * y + np.float32(c)
    return u * p


_RC = 128


@functools.lru_cache(maxsize=1)
def _sc_gather_fn():
    mesh = plsc.VectorSubcoreMesh(core_axis_name="c", subcore_axis_name="s")

    @functools.partial(
        pl.kernel,
        mesh=mesh,
        out_type=jax.ShapeDtypeStruct((_B,), jnp.float32),
        scratch_types=[
            pltpu.VMEM((_B,), jnp.int32),
            pltpu.VMEM((_B,), jnp.float32),
            pltpu.SemaphoreType.DMA,
        ],
    )
    def _sc_gather(t_hbm, a_hbm, out_hbm, idx_v, vals_v, sem):
        @pl.when((lax.axis_index("c") == 0) & (lax.axis_index("s") == 0))
        def _():
            pltpu.sync_copy(t_hbm, idx_v)
            pltpu.async_copy(a_hbm.at[idx_v], vals_v, sem).wait()
            pltpu.sync_copy(vals_v, out_hbm)

    return _sc_gather


def _noise_kernel(abar_ref, x0_ref, xt_ref, eps_ref):
    b = pl.program_id(0)
    rb = pl.program_id(1)
    abar = abar_ref[b]
    s0 = jnp.sqrt(abar)
    s1 = jnp.sqrt(np.float32(1.0) - abar)
    base1 = (b * np.int32(_H * _W) + rb * np.int32(_RB * _W)
             + np.int32(int(_K1))).astype(jnp.uint32)
    row = jax.lax.broadcasted_iota(jnp.uint32, (1, _RC, _W), 1)
    col = jax.lax.broadcasted_iota(jnp.uint32, (1, _RC, _W), 2)
    local = row * np.uint32(_W) + col

    def body(c, _):
        sl = pl.ds(c * _RC, _RC)
        idx = (base1 + (c * np.int32(_RC * _W)).astype(jnp.uint32)) + local
        eps = _std_normal(idx)
        eps_ref[:, sl, :] = eps
        xt_ref[:, sl, :] = s0 * x0_ref[:, sl, :] + s1 * eps
        return _

    jax.lax.fori_loop(0, _RB // _RC, body, 0, unroll=False)


def kernel(x0, t, alphas_cumprod):
    abar = _sc_gather_fn()(t, alphas_cumprod)
    blk = pl.BlockSpec((1, _RB, _W), lambda b, r: (b, r, 0))
    out = jax.ShapeDtypeStruct((_B, _H, _W), jnp.float32)
    x_t, eps = pl.pallas_call(
        _noise_kernel,
        grid=_GRID,
        in_specs=[
            pl.BlockSpec(memory_space=pltpu.SMEM),
            blk,
        ],
        out_specs=[blk, blk],
        out_shape=[out, out],
    )(abar, x0)
    return (x_t, eps)


# speedup vs baseline: 1.0265x; 1.0135x over previous
import functools

import numpy as np
import jax
import jax.numpy as jnp
from jax import lax
from jax.experimental import pallas as pl
from jax.experimental.pallas import tpu as pltpu
from jax.experimental.pallas import tpu_sc as plsc

_K0 = np.uint32(0)
_K1 = np.uint32(1)
_K2 = np.uint32(int(_K0) ^ int(_K1) ^ 0x1BD11BDA)
_ROT = ((13, 15, 26, 6), (17, 29, 16, 24))

_B, _H, _W = 64, 1024, 512
_RB = 512
_GRID = (_B, _H // _RB)

_U_LO = np.float32(np.nextafter(np.float32(-1.0), np.float32(0.0)))
_C = np.float32(0.62147385)
_PG = (462.2859191894531, -179.91043090820312, -48.60850524902344,
       3.161179304122925, 22.437999725341797, -14.577473640441895,
       7.130136013031006, -2.9441678524017334,
       1.802288293838501)


def _threefry_bits(x1):
    ks = (_K0, _K1, _K2)
    x0 = x1
    x1 = ((x1 << np.uint32(13)) | (x1 >> np.uint32(19))) ^ x0
    for i in range(5):
        for r in _ROT[i % 2][1 if i == 0 else 0:]:
            x0 = x0 + x1
            x1 = ((x1 << np.uint32(r)) | (x1 >> np.uint32(32 - r))) ^ x0
        if int(ks[(i + 1) % 3]) != 0:
            x0 = x0 + ks[(i + 1) % 3]
        x1 = x1 + np.uint32((int(ks[(i + 2) % 3]) + i + 1) & 0xFFFFFFFF)
    return x0 ^ x1


def _std_normal(idx):
    bits = _threefry_bits(idx)
    f = jax.lax.bitcast_convert_type(
        (bits >> np.uint32(9)) | np.uint32(0x40000000), jnp.float32)
    u = jnp.maximum(_U_LO, f - np.float32(3.0))
    z = jax.lax.rsqrt(np.float32(1.0) - jnp.log(np.float32(1.0) - u * u))
    y = z - _C
    p = jnp.full_like(y, _PG[0])
    for c in _PG[1:]:
        p = p * y + np.float32(c)
    return u * p


_RC = 128


@functools.lru_cache(maxsize=1)
def _sc_gather_fn():
    mesh = plsc.VectorSubcoreMesh(core_axis_name="c", subcore_axis_name="s")

    @functools.partial(
        pl.kernel,
        mesh=mesh,
        out_type=jax.ShapeDtypeStruct((_B,), jnp.float32),
        scratch_types=[
            pltpu.VMEM((_B,), jnp.int32),
            pltpu.VMEM((_B,), jnp.float32),
            pltpu.SemaphoreType.DMA,
        ],
    )
    def _sc_gather(t_hbm, a_hbm, out_hbm, idx_v, vals_v, sem):
        @pl.when((lax.axis_index("c") == 0) & (lax.axis_index("s") == 0))
        def _():
            pltpu.sync_copy(t_hbm, idx_v)
            pltpu.async_copy(a_hbm.at[idx_v], vals_v, sem).wait()
            pltpu.sync_copy(vals_v, out_hbm)

    return _sc_gather


def _noise_kernel(abar_ref, x0_ref, xt_ref, eps_ref):
    b = pl.program_id(0)
    rb = pl.program_id(1)
    abar = abar_ref[b]
    s0 = jnp.sqrt(abar)
    s1 = jnp.sqrt(np.float32(1.0) - abar)
    base1 = (b * np.int32(_H * _W) + rb * np.int32(_RB * _W)
             + np.int32(int(_K1))).astype(jnp.uint32)
    row = jax.lax.broadcasted_iota(jnp.uint32, (1, _RC, _W), 1)
    col = jax.lax.broadcasted_iota(jnp.uint32, (1, _RC, _W), 2)
    local = row * np.uint32(_W) + col

    def body(c, _):
        sl = pl.ds(c * _RC, _RC)
        idx = (base1 + (c * np.int32(_RC * _W)).astype(jnp.uint32)) + local
        eps = _std_normal(idx)
        eps_ref[:, sl, :] = eps
        xt_ref[:, sl, :] = s0 * x0_ref[:, sl, :] + s1 * eps
        return _

    jax.lax.fori_loop(0, _RB // _RC, body, 0, unroll=False)


def kernel(x0, t, alphas_cumprod):
    abar = _sc_gather_fn()(t, alphas_cumprod)
    blk = pl.BlockSpec((1, _RB, _W), lambda b, r: (b, r, 0))
    out = jax.ShapeDtypeStruct((_B, _H, _W), jnp.float32)
    x_t, eps = pl.pallas_call(
        _noise_kernel,
        grid=_GRID,
        in_specs=[
            pl.BlockSpec(memory_space=pltpu.SMEM),
            blk,
        ],
        out_specs=[blk, blk],
        out_shape=[out, out],
    )(abar, x0)
    return (x_t, eps)


# speedup vs baseline: 1.0328x; 1.0062x over previous
import functools

import numpy as np
import jax
import jax.numpy as jnp
from jax import lax
from jax.experimental import pallas as pl
from jax.experimental.pallas import tpu as pltpu
from jax.experimental.pallas import tpu_sc as plsc

_K0 = np.uint32(0)
_K1 = np.uint32(1)
_K2 = np.uint32(int(_K0) ^ int(_K1) ^ 0x1BD11BDA)
_ROT = ((13, 15, 26, 6), (17, 29, 16, 24))

_B, _H, _W = 64, 1024, 512
_RB = 1024
_GRID = (_B, _H // _RB)

_U_LO = np.float32(np.nextafter(np.float32(-1.0), np.float32(0.0)))
_C = np.float32(0.62147385)
_PG = (462.2859191894531, -179.91043090820312, -48.60850524902344,
       3.161179304122925, 22.437999725341797, -14.577473640441895,
       7.130136013031006, -2.9441678524017334,
       1.802288293838501)


def _threefry_bits(x1):
    ks = (_K0, _K1, _K2)
    x0 = x1
    x1 = ((x1 << np.uint32(13)) | (x1 >> np.uint32(19))) ^ x0
    for i in range(5):
        for r in _ROT[i % 2][1 if i == 0 else 0:]:
            x0 = x0 + x1
            x1 = ((x1 << np.uint32(r)) | (x1 >> np.uint32(32 - r))) ^ x0
        if int(ks[(i + 1) % 3]) != 0:
            x0 = x0 + ks[(i + 1) % 3]
        x1 = x1 + np.uint32((int(ks[(i + 2) % 3]) + i + 1) & 0xFFFFFFFF)
    return x0 ^ x1


def _std_normal(idx):
    bits = _threefry_bits(idx)
    f = jax.lax.bitcast_convert_type(
        (bits >> np.uint32(9)) | np.uint32(0x40000000), jnp.float32)
    u = jnp.maximum(_U_LO, f - np.float32(3.0))
    z = jax.lax.rsqrt(np.float32(1.0) - jnp.log(np.float32(1.0) - u * u))
    y = z - _C
    p = jnp.full_like(y, _PG[0])
    for c in _PG[1:]:
        p = p * y + np.float32(c)
    return u * p


_RC = 128


@functools.lru_cache(maxsize=1)
def _sc_gather_fn():
    mesh = plsc.VectorSubcoreMesh(core_axis_name="c", subcore_axis_name="s")

    @functools.partial(
        pl.kernel,
        mesh=mesh,
        out_type=jax.ShapeDtypeStruct((_B,), jnp.float32),
        scratch_types=[
            pltpu.VMEM((_B,), jnp.int32),
            pltpu.VMEM((_B,), jnp.float32),
            pltpu.SemaphoreType.DMA,
        ],
    )
    def _sc_gather(t_hbm, a_hbm, out_hbm, idx_v, vals_v, sem):
        @pl.when((lax.axis_index("c") == 0) & (lax.axis_index("s") == 0))
        def _():
            pltpu.sync_copy(t_hbm, idx_v)
            pltpu.async_copy(a_hbm.at[idx_v], vals_v, sem).wait()
            pltpu.sync_copy(vals_v, out_hbm)

    return _sc_gather


def _noise_kernel(abar_ref, x0_ref, xt_ref, eps_ref):
    b = pl.program_id(0)
    rb = pl.program_id(1)
    abar = abar_ref[b]
    s0 = jnp.sqrt(abar)
    s1 = jnp.sqrt(np.float32(1.0) - abar)
    base1 = (b * np.int32(_H * _W) + rb * np.int32(_RB * _W)
             + np.int32(int(_K1))).astype(jnp.uint32)
    row = jax.lax.broadcasted_iota(jnp.uint32, (1, _RC, _W), 1)
    col = jax.lax.broadcasted_iota(jnp.uint32, (1, _RC, _W), 2)
    local = row * np.uint32(_W) + col

    def body(c, _):
        sl = pl.ds(c * _RC, _RC)
        idx = (base1 + (c * np.int32(_RC * _W)).astype(jnp.uint32)) + local
        eps = _std_normal(idx)
        eps_ref[:, sl, :] = eps
        xt_ref[:, sl, :] = s0 * x0_ref[:, sl, :] + s1 * eps
        return _

    jax.lax.fori_loop(0, _RB // _RC, body, 0, unroll=False)


def kernel(x0, t, alphas_cumprod):
    abar = _sc_gather_fn()(t, alphas_cumprod)
    blk = pl.BlockSpec((1, _RB, _W), lambda b, r: (b, r, 0))
    out = jax.ShapeDtypeStruct((_B, _H, _W), jnp.float32)
    x_t, eps = pl.pallas_call(
        _noise_kernel,
        grid=_GRID,
        in_specs=[
            pl.BlockSpec(memory_space=pltpu.SMEM),
            blk,
        ],
        out_specs=[blk, blk],
        out_shape=[out, out],
    )(abar, x0)
    return (x_t, eps)


# speedup vs baseline: 1.0511x; 1.0176x over previous
import functools

import numpy as np
import jax
import jax.numpy as jnp
from jax import lax
from jax.experimental import pallas as pl
from jax.experimental.pallas import tpu as pltpu
from jax.experimental.pallas import tpu_sc as plsc

_K0 = np.uint32(0)
_K1 = np.uint32(1)
_K2 = np.uint32(int(_K0) ^ int(_K1) ^ 0x1BD11BDA)
_ROT = ((13, 15, 26, 6), (17, 29, 16, 24))

_B, _H, _W = 64, 1024, 512
_RB = 1024
_GRID = (_B, _H // _RB)

_U_LO = np.float32(np.nextafter(np.float32(-1.0), np.float32(0.0)))
_C = np.float32(0.62147385)
_PG = (462.2859191894531, -179.91043090820312, -48.60850524902344,
       3.161179304122925, 22.437999725341797, -14.577473640441895,
       7.130136013031006, -2.9441678524017334,
       1.802288293838501)


def _threefry_bits(x1):
    ks = (_K0, _K1, _K2)
    x0 = x1
    x1 = ((x1 << np.uint32(13)) | (x1 >> np.uint32(19))) ^ x0
    for i in range(5):
        for r in _ROT[i % 2][1 if i == 0 else 0:]:
            x0 = x0 + x1
            x1 = ((x1 << np.uint32(r)) | (x1 >> np.uint32(32 - r))) ^ x0
        if int(ks[(i + 1) % 3]) != 0:
            x0 = x0 + ks[(i + 1) % 3]
        x1 = x1 + np.uint32((int(ks[(i + 2) % 3]) + i + 1) & 0xFFFFFFFF)
    return x0 ^ x1


def _std_normal(idx):
    bits = _threefry_bits(idx)
    f = jax.lax.bitcast_convert_type(
        (bits >> np.uint32(9)) | np.uint32(0x40000000), jnp.float32)
    u = jnp.maximum(_U_LO, f - np.float32(3.0))
    z = jax.lax.rsqrt(np.float32(1.0) - jnp.log(np.float32(1.0) - u * u))
    y = z - _C
    p = jnp.full_like(y, _PG[0])
    for c in _PG[1:]:
        p = p * y + np.float32(c)
    return u * p


_RC = 128


@functools.lru_cache(maxsize=1)
def _sc_gather_fn():
    mesh = plsc.VectorSubcoreMesh(core_axis_name="c", subcore_axis_name="s")

    @functools.partial(
        pl.kernel,
        mesh=mesh,
        out_type=jax.ShapeDtypeStruct((_B,), jnp.float32),
        scratch_types=[
            pltpu.VMEM((_B,), jnp.int32),
            pltpu.VMEM((_B,), jnp.float32),
            pltpu.SemaphoreType.DMA,
        ],
    )
    def _sc_gather(t_hbm, a_hbm, out_hbm, idx_v, vals_v, sem):
        @pl.when((lax.axis_index("c") == 0) & (lax.axis_index("s") == 0))
        def _():
            pltpu.sync_copy(t_hbm, idx_v)
            pltpu.async_copy(a_hbm.at[idx_v], vals_v, sem).wait()
            pltpu.sync_copy(vals_v, out_hbm)

    return _sc_gather


def _noise_kernel(abar_ref, x0_ref, xt_ref, eps_ref):
    b = pl.program_id(0)
    rb = pl.program_id(1)
    abar = abar_ref[b]
    s0 = jnp.sqrt(abar)
    s1 = jnp.sqrt(np.float32(1.0) - abar)
    base1 = (b * np.int32(_H * _W) + rb * np.int32(_RB * _W)
             + np.int32(int(_K1))).astype(jnp.uint32)
    row = jax.lax.broadcasted_iota(jnp.uint32, (1, _RC, _W), 1)
    col = jax.lax.broadcasted_iota(jnp.uint32, (1, _RC, _W), 2)
    local = row * np.uint32(_W) + col

    def body(c, _):
        sl = pl.ds(c * _RC, _RC)
        idx = (base1 + (c * np.int32(_RC * _W)).astype(jnp.uint32)) + local
        eps = _std_normal(idx)
        eps_ref[:, sl, :] = eps
        xt_ref[:, sl, :] = s0 * x0_ref[:, sl, :] + s1 * eps
        return _

    jax.lax.fori_loop(0, _RB // _RC, body, 0, unroll=4)


def kernel(x0, t, alphas_cumprod):
    abar = _sc_gather_fn()(t, alphas_cumprod)
    blk = pl.BlockSpec((1, _RB, _W), lambda b, r: (b, r, 0))
    out = jax.ShapeDtypeStruct((_B, _H, _W), jnp.float32)
    x_t, eps = pl.pallas_call(
        _noise_kernel,
        grid=_GRID,
        in_specs=[
            pl.BlockSpec(memory_space=pltpu.SMEM),
            blk,
        ],
        out_specs=[blk, blk],
        out_shape=[out, out],
    )(abar, x0)
    return (x_t, eps)
